# baseline (speedup 1.0000x reference)
import functools

import jax
import jax.numpy as jnp
from jax import lax
from jax.experimental import pallas as pl
from jax.experimental.pallas import tpu as pltpu
from jax.experimental.pallas import tpu_sc as plsc

_CLS = 32000
_SMOOTH = 0.1
_N = 2048
_A = _SMOOTH / (_CLS - 1)
_B = 1.0 - _SMOOTH - _A

_R = 128
_NB = _N // _R
_CH = 3200

_NC = 2
_NS = 16
_NW = _NC * _NS
_BW = _N // _NW
_L = 16


def _tc_body(x_ref, pt_ref, o_ref, acc_ref):
    i = pl.program_id(0)

    x = x_ref[:, 0:_CH]
    m = jnp.max(x, axis=1, keepdims=True)
    se = jnp.sum(jnp.exp(x - m), axis=1, keepdims=True)
    s = jnp.sum(x, axis=1, keepdims=True)
    for c in range(_CH, _CLS, _CH):
        x = x_ref[:, c:c + _CH]
        tm = jnp.max(x, axis=1, keepdims=True)
        mn = jnp.maximum(m, tm)
        se = se * jnp.exp(m - mn) + jnp.sum(jnp.exp(x - mn), axis=1,
                                            keepdims=True)
        m = mn
        s = s + jnp.sum(x, axis=1, keepdims=True)

    lse = m + jnp.log(se)
    partial = jnp.sum(lse - _A * s).reshape(1, 1)

    @pl.when(i == 0)
    def _():
        acc_ref[:, :] = jnp.zeros((1, 1), jnp.float32)

    acc_ref[:, :] = acc_ref[:, :] + partial

    @pl.when(i == _NB - 1)
    def _():
        pt_sum = jnp.sum(pt_ref[:, :]).reshape(1, 1)
        o_ref[:, :] = (acc_ref[:, :] - _B * pt_sum) * (1.0 / _N)


def _sc_body(pred_hbm, tgt_hbm, out_hbm, tgt_v, val_v, psum_v, sem):
    wid = lax.axis_index("s") * _NC + lax.axis_index("c")
    base = wid * _BW
    pltpu.sync_copy(tgt_hbm.at[pl.ds(base, _BW)], tgt_v)
    lane = lax.iota(jnp.int32, _L)
    copies = []
    for j in range(_BW):
        t16 = tgt_v[pl.ds((j // _L) * _L, _L)]
        tj = jnp.sum(jnp.where(lane == (j % _L), t16, 0))
        cb = (tj // _L) * _L
        copies.append(
            pltpu.async_copy(pred_hbm.at[base + j, pl.ds(cb, _L)],
                             val_v.at[j], sem))
    for c in copies:
        c.wait()
    acc = jnp.zeros((_L,), jnp.float32)
    for j4 in range(_BW // _L):
        ridx = j4 * _L + lane
        cidx = tgt_v[pl.ds(j4 * _L, _L)] & (_L - 1)
        acc = acc + plsc.load_gather(val_v, [ridx, cidx])
    psum_v[...] = acc
    pltpu.sync_copy(psum_v, out_hbm.at[wid])


@functools.partial(
    pl.kernel,
    mesh=plsc.VectorSubcoreMesh(core_axis_name="c", subcore_axis_name="s"),
    compiler_params=pltpu.CompilerParams(needs_layout_passes=False),
    out_type=jax.ShapeDtypeStruct((_NW, _L), jnp.float32),
    scratch_types=[
        pltpu.VMEM((_BW,), jnp.int32),
        pltpu.VMEM((_BW, _L), jnp.float32),
        pltpu.VMEM((_L,), jnp.float32),
        pltpu.SemaphoreType.DMA,
    ],
)
def _sc_gather(pred_hbm, tgt_hbm, out_hbm, tgt_v, val_v, psum_v, sem):
    _sc_body(pred_hbm, tgt_hbm, out_hbm, tgt_v, val_v, psum_v, sem)


@jax.jit
def kernel(pred, target):
    pt_partials = _sc_gather(pred, target.astype(jnp.int32))
    tc_out = pl.pallas_call(
        _tc_body,
        grid=(_NB,),
        in_specs=[
            pl.BlockSpec((_R, _CLS), lambda i: (i, 0)),
            pl.BlockSpec((_NW, _L), lambda i: (0, 0)),
        ],
        out_specs=pl.BlockSpec((1, 1), lambda i: (0, 0)),
        out_shape=jax.ShapeDtypeStruct((1, 1), jnp.float32),
        scratch_shapes=[pltpu.VMEM((1, 1), jnp.float32)],
    )(pred, pt_partials)
    return tc_out[0, 0]


# speedup vs baseline: 1.0661x; 1.0661x over previous
import functools

import jax
import jax.numpy as jnp
from jax import lax
from jax.experimental import pallas as pl
from jax.experimental.pallas import tpu as pltpu
from jax.experimental.pallas import tpu_sc as plsc

_CLS = 32000
_SMOOTH = 0.1
_N = 2048
_A = _SMOOTH / (_CLS - 1)
_B = 1.0 - _SMOOTH - _A

_R = 128
_NB = _N // _R
_CH = 3200

_NC = 2
_NS = 16
_NW = _NC * _NS
_BW = _N // _NW
_L = 16


def _tc_body(x_ref, o_ref, acc_ref):
    i = pl.program_id(0)

    x = x_ref[:, 0:_CH]
    m = jnp.max(x, axis=1, keepdims=True)
    se = jnp.sum(jnp.exp(x - m), axis=1, keepdims=True)
    s = jnp.sum(x, axis=1, keepdims=True)
    for c in range(_CH, _CLS, _CH):
        x = x_ref[:, c:c + _CH]
        tm = jnp.max(x, axis=1, keepdims=True)
        mn = jnp.maximum(m, tm)
        se = se * jnp.exp(m - mn) + jnp.sum(jnp.exp(x - mn), axis=1,
                                            keepdims=True)
        m = mn
        s = s + jnp.sum(x, axis=1, keepdims=True)

    lse = m + jnp.log(se)
    partial = jnp.sum(lse - _A * s).reshape(1, 1)

    @pl.when(i == 0)
    def _():
        acc_ref[:, :] = jnp.zeros((1, 1), jnp.float32)

    acc_ref[:, :] = acc_ref[:, :] + partial

    @pl.when(i == _NB - 1)
    def _():
        o_ref[:, :] = acc_ref[:, :]


def _sc_body(pred_hbm, tgt_hbm, out_hbm, tgt_v, val_v, psum_v, sem):
    wid = lax.axis_index("s") * _NC + lax.axis_index("c")
    base = wid * _BW
    pltpu.sync_copy(tgt_hbm.at[pl.ds(base, _BW)], tgt_v)
    lane = lax.iota(jnp.int32, _L)
    copies = []
    for j in range(_BW):
        t16 = tgt_v[pl.ds((j // _L) * _L, _L)]
        tj = jnp.sum(jnp.where(lane == (j % _L), t16, 0))
        cb = (tj // _L) * _L
        copies.append(
            pltpu.async_copy(pred_hbm.at[base + j, pl.ds(cb, _L)],
                             val_v.at[j], sem))
    for c in copies:
        c.wait()
    acc = jnp.zeros((_L,), jnp.float32)
    for j4 in range(_BW // _L):
        ridx = j4 * _L + lane
        cidx = tgt_v[pl.ds(j4 * _L, _L)] & (_L - 1)
        acc = acc + plsc.load_gather(val_v, [ridx, cidx])
    psum_v[...] = acc
    pltpu.sync_copy(psum_v, out_hbm.at[wid])


@functools.partial(
    pl.kernel,
    mesh=plsc.VectorSubcoreMesh(core_axis_name="c", subcore_axis_name="s"),
    compiler_params=pltpu.CompilerParams(needs_layout_passes=False),
    out_type=jax.ShapeDtypeStruct((_NW, _L), jnp.float32),
    scratch_types=[
        pltpu.VMEM((_BW,), jnp.int32),
        pltpu.VMEM((_BW, _L), jnp.float32),
        pltpu.VMEM((_L,), jnp.float32),
        pltpu.SemaphoreType.DMA,
    ],
)
def _sc_gather(pred_hbm, tgt_hbm, out_hbm, tgt_v, val_v, psum_v, sem):
    _sc_body(pred_hbm, tgt_hbm, out_hbm, tgt_v, val_v, psum_v, sem)


@jax.jit
def kernel(pred, target):
    tc_out = pl.pallas_call(
        _tc_body,
        grid=(_NB,),
        in_specs=[pl.BlockSpec((_R, _CLS), lambda i: (i, 0))],
        out_specs=pl.BlockSpec((1, 1), lambda i: (0, 0)),
        out_shape=jax.ShapeDtypeStruct((1, 1), jnp.float32),
        scratch_shapes=[pltpu.VMEM((1, 1), jnp.float32)],
    )(pred)
    pt_partials = _sc_gather(pred, target.astype(jnp.int32))
    return (tc_out[0, 0] - _B * jnp.sum(pt_partials)) * (1.0 / _N)


# speedup vs baseline: 1.0662x; 1.0001x over previous
import functools

import jax
import jax.numpy as jnp
from jax import lax
from jax.experimental import pallas as pl
from jax.experimental.pallas import tpu as pltpu
from jax.experimental.pallas import tpu_sc as plsc

_CLS = 32000
_SMOOTH = 0.1
_N = 2048
_A = _SMOOTH / (_CLS - 1)
_B = 1.0 - _SMOOTH - _A

_R = 128
_NB = _N // _R
_CH = 3200

_NC = 2
_NS = 16
_NW = _NC * _NS
_BW = _N // _NW
_L = 16


def _tc_body(x_ref, o_ref, acc_ref):
    i = pl.program_id(0)

    x = x_ref[:, 0:_CH]
    m = jnp.max(x, axis=1, keepdims=True)
    se = jnp.sum(jnp.exp(x - m), axis=1, keepdims=True)
    s = jnp.sum(x, axis=1, keepdims=True)
    for c in range(_CH, _CLS, _CH):
        x = x_ref[:, c:c + _CH]
        tm = jnp.max(x, axis=1, keepdims=True)
        mn = jnp.maximum(m, tm)
        se = se * jnp.exp(m - mn) + jnp.sum(jnp.exp(x - mn), axis=1,
                                            keepdims=True)
        m = mn
        s = s + jnp.sum(x, axis=1, keepdims=True)

    lse = m + jnp.log(se)
    partial = jnp.sum(lse - _A * s).reshape(1, 1)

    @pl.when(i == 0)
    def _():
        acc_ref[:, :] = jnp.zeros((1, 1), jnp.float32)

    acc_ref[:, :] = acc_ref[:, :] + partial

    @pl.when(i == _NB - 1)
    def _():
        o_ref[:, :] = acc_ref[:, :]


def _sc_body(pred_hbm, tgt_hbm, out_hbm, tgt_v, val_v, psum_v, sem):
    wid = lax.axis_index("s") * _NC + lax.axis_index("c")
    base = wid * _BW
    pltpu.sync_copy(tgt_hbm.at[pl.ds(base, _BW)], tgt_v)
    lane = lax.iota(jnp.int32, _L)
    copies = []
    for j in range(_BW):
        t16 = tgt_v[pl.ds((j // _L) * _L, _L)]
        tj = t16[j % _L]
        cb = (tj // _L) * _L
        copies.append(
            pltpu.async_copy(pred_hbm.at[base + j, pl.ds(cb, _L)],
                             val_v.at[j], sem))
    for c in copies:
        c.wait()
    acc = jnp.zeros((_L,), jnp.float32)
    for j4 in range(_BW // _L):
        ridx = j4 * _L + lane
        cidx = tgt_v[pl.ds(j4 * _L, _L)] & (_L - 1)
        acc = acc + plsc.load_gather(val_v, [ridx, cidx])
    psum_v[...] = acc
    pltpu.sync_copy(psum_v, out_hbm.at[wid])


@functools.partial(
    pl.kernel,
    mesh=plsc.VectorSubcoreMesh(core_axis_name="c", subcore_axis_name="s"),
    compiler_params=pltpu.CompilerParams(needs_layout_passes=False),
    out_type=jax.ShapeDtypeStruct((_NW, _L), jnp.float32),
    scratch_types=[
        pltpu.VMEM((_BW,), jnp.int32),
        pltpu.VMEM((_BW, _L), jnp.float32),
        pltpu.VMEM((_L,), jnp.float32),
        pltpu.SemaphoreType.DMA,
    ],
)
def _sc_gather(pred_hbm, tgt_hbm, out_hbm, tgt_v, val_v, psum_v, sem):
    _sc_body(pred_hbm, tgt_hbm, out_hbm, tgt_v, val_v, psum_v, sem)


@jax.jit
def kernel(pred, target):
    tc_out = pl.pallas_call(
        _tc_body,
        grid=(_NB,),
        in_specs=[pl.BlockSpec((_R, _CLS), lambda i: (i, 0))],
        out_specs=pl.BlockSpec((1, 1), lambda i: (0, 0)),
        out_shape=jax.ShapeDtypeStruct((1, 1), jnp.float32),
        scratch_shapes=[pltpu.VMEM((1, 1), jnp.float32)],
    )(pred)
    pt_partials = _sc_gather(pred, target.astype(jnp.int32))
    return (tc_out[0, 0] - _B * jnp.sum(pt_partials)) * (1.0 / _N)
